# baseline (speedup 1.0000x reference)
import functools

import jax
import jax.numpy as jnp
from jax import lax
from jax.experimental import pallas as pl
from jax.experimental.pallas import tpu as pltpu
from jax.experimental.pallas import tpu_sc as plsc

B, C, H, W = 2, 384, 224, 224
HW = H * W
N = B * HW

NC, NS, L = 2, 16, 16
NW = NC * NS
PPW = N // NW
CHUNKS = PPW // L
CPR = W // L
CPL = C // L

_mesh = plsc.VectorSubcoreMesh(core_axis_name="c", subcore_axis_name="s")


@functools.partial(
    pl.kernel,
    out_type=jax.ShapeDtypeStruct((N, C), jnp.float32),
    mesh=_mesh,
    compiler_params=pltpu.CompilerParams(needs_layout_passes=False),
    scratch_types=[
        pltpu.VMEM((PPW,), jnp.float32),
        pltpu.VMEM((PPW,), jnp.float32),
        [pltpu.VMEM((4 * L, C), jnp.float32)] * 2,
        [pltpu.VMEM((L, C), jnp.float32)] * 2,
        [pltpu.VMEM((L,), jnp.float32)] * 2,
        [pltpu.VMEM((L,), jnp.float32)] * 2,
        [pltpu.VMEM((4 * L,), jnp.int32)] * 2,
        [pltpu.SemaphoreType.DMA] * 2,
        [pltpu.SemaphoreType.DMA] * 2,
    ],
)
def _resample(img, fx_all, fy_all, out, fx_v, fy_v, gbufs, obufs, axbs, aybs,
              ibufs, gsems, osems):
    wid = lax.axis_index("s") * NC + lax.axis_index("c")
    p0 = wid * PPW
    img_base = (p0 // HW) * HW
    ybase = (p0 % HW) // W
    pltpu.sync_copy(fx_all.at[pl.ds(p0, PPW)], fx_v)
    pltpu.sync_copy(fy_all.at[pl.ds(p0, PPW)], fy_v)
    xiota = lax.iota(jnp.int32, L).astype(jnp.float32)

    def stage(k, s):
        y = ybase + k // CPR
        xs = (k % CPR) * L
        fxv = fx_v[pl.ds(k * L, L)]
        fyv = fy_v[pl.ds(k * L, L)]
        xc = jnp.clip(xs * 1.0 + xiota + fxv, 0.0, W - 1.0)
        yc = jnp.clip(y * 1.0 + fyv, 0.0, H - 1.0)
        x0 = jnp.minimum(xc.astype(jnp.int32), W - 2)
        y0 = jnp.minimum(yc.astype(jnp.int32), H - 2)
        axbs[s][...] = xc - x0.astype(jnp.float32)
        aybs[s][...] = yc - y0.astype(jnp.float32)
        row00 = img_base + y0 * W + x0
        i4 = lax.iota(jnp.int32, L) * 4
        plsc.store_scatter(ibufs[s], [i4], row00)
        plsc.store_scatter(ibufs[s], [i4 + 1], row00 + 1)
        plsc.store_scatter(ibufs[s], [i4 + 2], row00 + W)
        plsc.store_scatter(ibufs[s], [i4 + 3], row00 + W + 1)
        pltpu.async_copy(img.at[ibufs[s]], gbufs[s], gsems[s])

    def step(k, s):
        pltpu.make_async_copy(img.at[pl.ds(0, 4 * L)], gbufs[s], gsems[s]).wait()
        @pl.when(k >= 2)
        def _():
            pltpu.make_async_copy(obufs[s], out.at[pl.ds(p0, L)], osems[s]).wait()

        gbuf, obuf = gbufs[s], obufs[s]

        @pl.loop(0, L)
        def _px(i):
            iv = jnp.zeros((L,), jnp.int32) + i
            a_x = plsc.load_gather(axbs[s], [iv])
            a_y = plsc.load_gather(aybs[s], [iv])
            w00 = (1.0 - a_x) * (1.0 - a_y)
            w01 = a_x * (1.0 - a_y)
            w10 = (1.0 - a_x) * a_y
            w11 = a_x * a_y
            for j in range(CPL):
                sl = pl.ds(j * L, L)
                obuf[i, sl] = (gbuf[4 * i, sl] * w00
                               + gbuf[4 * i + 1, sl] * w01
                               + gbuf[4 * i + 2, sl] * w10
                               + gbuf[4 * i + 3, sl] * w11)

        pltpu.async_copy(obuf, out.at[pl.ds(p0 + k * L, L)], osems[s])

        @pl.when(k + 2 < CHUNKS)
        def _():
            stage(k + 2, s)

    stage(0, 0)
    stage(1, 1)

    @pl.loop(0, CHUNKS, step=2)
    def _chunk(k):
        step(k, 0)
        step(k + 1, 1)

    pltpu.make_async_copy(obufs[0], out.at[pl.ds(p0, L)], osems[0]).wait()
    pltpu.make_async_copy(obufs[1], out.at[pl.ds(p0, L)], osems[1]).wait()


def kernel(input1, input2):
    img = jnp.transpose(input1, (0, 2, 3, 1)).reshape(N, C)
    fx = input2[:, 0].reshape(-1)
    fy = input2[:, 1].reshape(-1)
    out_cl = _resample(img, fx, fy)
    return jnp.transpose(out_cl.reshape(B, H, W, C), (0, 3, 1, 2))


# speedup vs baseline: 2.1702x; 2.1702x over previous
import functools

import jax
import jax.numpy as jnp
from jax import lax
from jax.experimental import pallas as pl
from jax.experimental.pallas import tpu as pltpu
from jax.experimental.pallas import tpu_sc as plsc

B, C, H, W = 2, 384, 224, 224
HW = H * W
N = B * HW

NC, NS, L = 2, 16, 16
NW = NC * NS
PPW = N // NW
CHUNKS = PPW // L
CPR = W // L
CPL = C // L

_mesh = plsc.VectorSubcoreMesh(core_axis_name="c", subcore_axis_name="s")


@functools.partial(
    pl.kernel,
    out_type=jax.ShapeDtypeStruct((N, C), jnp.float32),
    mesh=_mesh,
    compiler_params=pltpu.CompilerParams(needs_layout_passes=False),
    scratch_types=[
        pltpu.VMEM((PPW,), jnp.float32),
        pltpu.VMEM((PPW,), jnp.float32),
        [pltpu.VMEM((4 * L, C), jnp.float32)] * 2,
        [pltpu.VMEM((L, C), jnp.float32)] * 2,
        [pltpu.VMEM((L,), jnp.float32)] * 2,
        [pltpu.VMEM((L,), jnp.float32)] * 2,
        [pltpu.VMEM((4 * L,), jnp.int32)] * 2,
        [pltpu.SemaphoreType.DMA] * 2,
        [pltpu.SemaphoreType.DMA] * 2,
    ],
)
def _resample(img, fx_all, fy_all, out, fx_v, fy_v, gbufs, obufs, axbs, aybs,
              ibufs, gsems, osems):
    wid = lax.axis_index("s") * NC + lax.axis_index("c")
    p0 = wid * PPW
    img_base = (p0 // HW) * HW
    ybase = (p0 % HW) // W
    pltpu.sync_copy(fx_all.at[pl.ds(p0, PPW)], fx_v)
    pltpu.sync_copy(fy_all.at[pl.ds(p0, PPW)], fy_v)
    xiota = lax.iota(jnp.int32, L).astype(jnp.float32)

    def stage(k, s):
        y = ybase + k // CPR
        xs = (k % CPR) * L
        fxv = fx_v[pl.ds(k * L, L)]
        fyv = fy_v[pl.ds(k * L, L)]
        xc = jnp.clip(xs * 1.0 + xiota + fxv, 0.0, W - 1.0)
        yc = jnp.clip(y * 1.0 + fyv, 0.0, H - 1.0)
        x0 = jnp.minimum(xc.astype(jnp.int32), W - 2)
        y0 = jnp.minimum(yc.astype(jnp.int32), H - 2)
        axbs[s][...] = xc - x0.astype(jnp.float32)
        aybs[s][...] = yc - y0.astype(jnp.float32)
        row00 = img_base + y0 * W + x0
        ibufs[s][pl.ds(0 * L, L)] = row00
        ibufs[s][pl.ds(1 * L, L)] = row00 + 1
        ibufs[s][pl.ds(2 * L, L)] = row00 + W
        ibufs[s][pl.ds(3 * L, L)] = row00 + W + 1
        pltpu.async_copy(img.at[ibufs[s]], gbufs[s], gsems[s])

    def step(k, s):
        pltpu.make_async_copy(img.at[pl.ds(0, 4 * L)], gbufs[s], gsems[s]).wait()
        @pl.when(k >= 2)
        def _():
            pltpu.make_async_copy(obufs[s], out.at[pl.ds(p0, L)], osems[s]).wait()

        gbuf, obuf = gbufs[s], obufs[s]

        @pl.loop(0, L)
        def _px(i):
            iv = jnp.zeros((L,), jnp.int32) + i
            a_x = plsc.load_gather(axbs[s], [iv])
            a_y = plsc.load_gather(aybs[s], [iv])
            w00 = (1.0 - a_x) * (1.0 - a_y)
            w01 = a_x * (1.0 - a_y)
            w10 = (1.0 - a_x) * a_y
            w11 = a_x * a_y
            for j in range(CPL):
                sl = pl.ds(j * L, L)
                obuf[i, sl] = (gbuf[0 * L + i, sl] * w00
                               + gbuf[1 * L + i, sl] * w01
                               + gbuf[2 * L + i, sl] * w10
                               + gbuf[3 * L + i, sl] * w11)

        @pl.when(k + 2 < CHUNKS)
        def _():
            stage(k + 2, s)

        pltpu.async_copy(obuf, out.at[pl.ds(p0 + k * L, L)], osems[s])

    stage(0, 0)
    stage(1, 1)

    @pl.loop(0, CHUNKS, step=2)
    def _chunk(k):
        step(k, 0)
        step(k + 1, 1)

    pltpu.make_async_copy(obufs[0], out.at[pl.ds(p0, L)], osems[0]).wait()
    pltpu.make_async_copy(obufs[1], out.at[pl.ds(p0, L)], osems[1]).wait()


def kernel(input1, input2):
    img = jnp.transpose(input1, (0, 2, 3, 1)).reshape(N, C)
    fx = input2[:, 0].reshape(-1)
    fy = input2[:, 1].reshape(-1)
    out_cl = _resample(img, fx, fy)
    return jnp.transpose(out_cl.reshape(B, H, W, C), (0, 3, 1, 2))
